# baseline (speedup 1.0000x reference)
import jax
import jax.numpy as jnp
from jax import lax
from jax.experimental import pallas as pl
from jax.experimental.pallas import tpu as pltpu
from jax.experimental.pallas import tpu_sc as plsc

N = 524288
B = 64
NORM = 0.10538154

NC = 2
NS = 16
L = 16
NW = NC * NS
CHUNK = N // NW
STEPS = CHUNK // L
NCOMP = 6
ACC_WORDS = NCOMP * B * L
ROW = 8 * B


def _tec_body(pos_hbm, q_hbm, batch_hbm, out_hbm,
              x_v, y_v, z_v, q_v, b_v, acc_v, res_v,
              sem0, sem1, sem2, sem3, sem4):
    cid = lax.axis_index("c")
    sid = lax.axis_index("s")
    wid = cid * NS + sid
    base = wid * CHUNK

    c0 = pltpu.async_copy(pos_hbm.at[pl.ds(base, CHUNK)], x_v, sem0)
    c1 = pltpu.async_copy(pos_hbm.at[pl.ds(N + base, CHUNK)], y_v, sem1)
    c2 = pltpu.async_copy(pos_hbm.at[pl.ds(2 * N + base, CHUNK)], z_v, sem2)
    c3 = pltpu.async_copy(q_hbm.at[pl.ds(base, CHUNK)], q_v, sem3)
    c4 = pltpu.async_copy(batch_hbm.at[pl.ds(base, CHUNK)], b_v, sem4)

    lane = lax.iota(jnp.int32, L)
    zero = jnp.zeros((L,), jnp.float32)

    def zbody(i, _):
        w = i * (4 * L)
        acc_v[pl.ds(w, L)] = zero
        acc_v[pl.ds(w + L, L)] = zero
        acc_v[pl.ds(w + 2 * L, L)] = zero
        acc_v[pl.ds(w + 3 * L, L)] = zero
        return 0
    lax.fori_loop(0, ACC_WORDS // (4 * L), zbody, 0)

    c0.wait()
    c1.wait()
    c2.wait()
    c3.wait()
    c4.wait()

    @plsc.parallel_loop(0, CHUNK, step=L, unroll=4, carry=zero)
    def qsum(el, qsum):
        px = x_v[pl.ds(el, L)]
        py = y_v[pl.ds(el, L)]
        pz = z_v[pl.ds(el, L)]
        qv = q_v[pl.ds(el, L)]
        bv = b_v[pl.ds(el, L)]
        addr = bv * L + lane
        plsc.addupdate_scatter(acc_v, [addr], qv * px)
        plsc.addupdate_scatter(acc_v, [addr + (B * L)], qv * py)
        plsc.addupdate_scatter(acc_v, [addr + (2 * B * L)], qv * pz)
        plsc.addupdate_scatter(acc_v, [addr + (3 * B * L)], px)
        plsc.addupdate_scatter(acc_v, [addr + (4 * B * L)], py)
        plsc.addupdate_scatter(acc_v, [addr + (5 * B * L)], pz)
        return qsum + qv

    bgrp = lax.iota(jnp.int32, L) * L
    for c in range(NCOMP):
        for g in range(B // L):
            a0 = c * B * L + g * L * L
            s = zero
            for l in range(L):
                s = s + plsc.load_gather(acc_v, [bgrp + (a0 + l)])
            res_v[pl.ds(c * B + g * L, L)] = s
    qs = jnp.broadcast_to(jnp.sum(qsum), (L,))
    res_v[pl.ds(NCOMP * B, L)] = qs
    res_v[pl.ds(NCOMP * B + L, L)] = qs
    res_v[pl.ds(NCOMP * B + 2 * L, L)] = qs
    res_v[pl.ds(NCOMP * B + 3 * L, L)] = qs
    res_v[pl.ds(7 * B, L)] = zero
    res_v[pl.ds(7 * B + L, L)] = zero
    res_v[pl.ds(7 * B + 2 * L, L)] = zero
    res_v[pl.ds(7 * B + 3 * L, L)] = zero

    pltpu.sync_copy(res_v, out_hbm.at[wid])


def _partials(pos, q, batch):
    mesh = plsc.VectorSubcoreMesh(
        core_axis_name="c", subcore_axis_name="s", num_cores=NC, num_subcores=NS
    )
    return pl.kernel(
        _tec_body,
        out_type=jax.ShapeDtypeStruct((NW, ROW), jnp.float32),
        mesh=mesh,
        scratch_types=[
            pltpu.VMEM((CHUNK,), jnp.float32),
            pltpu.VMEM((CHUNK,), jnp.float32),
            pltpu.VMEM((CHUNK,), jnp.float32),
            pltpu.VMEM((CHUNK,), jnp.float32),
            pltpu.VMEM((CHUNK,), jnp.int32),
            pltpu.VMEM((ACC_WORDS,), jnp.float32),
            pltpu.VMEM((ROW,), jnp.float32),
            pltpu.SemaphoreType.DMA,
            pltpu.SemaphoreType.DMA,
            pltpu.SemaphoreType.DMA,
            pltpu.SemaphoreType.DMA,
            pltpu.SemaphoreType.DMA,
        ],
        compiler_params=pltpu.CompilerParams(needs_layout_passes=False),
    )(pos, q, batch)


def kernel(positions, q, batch, cell):
    del cell
    pos_t = jnp.einsum("ij,nj->in", jnp.eye(3, dtype=jnp.float32), positions,
                       preferred_element_type=jnp.float32)
    part = _partials(pos_t.reshape(3 * N), q, batch.astype(jnp.int32))
    t = part.sum(axis=0).reshape(8, B)
    pol = (t[0:3] - (t[6] * (1.0 / N)) * t[3:6]).T * NORM
    return pol


# speedup vs baseline: 1.0160x; 1.0160x over previous
import jax
import jax.numpy as jnp
from jax import lax
from jax.experimental import pallas as pl
from jax.experimental.pallas import tpu as pltpu
from jax.experimental.pallas import tpu_sc as plsc

N = 524288
B = 64
NORM = 0.10538154

NC = 2
NS = 16
L = 16
NW = NC * NS
CHUNK = N // NW
STEPS = CHUNK // L
NCOMP = 6
ACC_WORDS = NCOMP * B * L
ROW = 8 * B


def _tec_body(pos_hbm, q_hbm, batch_hbm, out_hbm,
              x_v, y_v, z_v, q_v, b_v, acc_v, res_v,
              sem0, sem1, sem2, sem3, sem4):
    cid = lax.axis_index("c")
    sid = lax.axis_index("s")
    wid = cid * NS + sid
    base = wid * CHUNK

    c0 = pltpu.async_copy(pos_hbm.at[pl.ds(base, CHUNK)], x_v, sem0)
    c1 = pltpu.async_copy(pos_hbm.at[pl.ds(N + base, CHUNK)], y_v, sem1)
    c2 = pltpu.async_copy(pos_hbm.at[pl.ds(2 * N + base, CHUNK)], z_v, sem2)
    c3 = pltpu.async_copy(q_hbm.at[pl.ds(base, CHUNK)], q_v, sem3)
    c4 = pltpu.async_copy(batch_hbm.at[pl.ds(base, CHUNK)], b_v, sem4)

    lane = lax.iota(jnp.int32, L)
    zero = jnp.zeros((L,), jnp.float32)

    def zbody(i, _):
        w = i * (4 * L)
        acc_v[pl.ds(w, L)] = zero
        acc_v[pl.ds(w + L, L)] = zero
        acc_v[pl.ds(w + 2 * L, L)] = zero
        acc_v[pl.ds(w + 3 * L, L)] = zero
        return 0
    lax.fori_loop(0, ACC_WORDS // (4 * L), zbody, 0)

    c0.wait()
    c1.wait()
    c2.wait()
    c3.wait()
    c4.wait()

    @plsc.parallel_loop(0, CHUNK, step=L, unroll=4, carry=zero)
    def qsum(el, qsum):
        px = x_v[pl.ds(el, L)]
        py = y_v[pl.ds(el, L)]
        pz = z_v[pl.ds(el, L)]
        qv = q_v[pl.ds(el, L)]
        bv = b_v[pl.ds(el, L)]
        addr = bv * L + lane
        plsc.addupdate_scatter(acc_v, [addr], qv * px)
        plsc.addupdate_scatter(acc_v, [addr + (B * L)], qv * py)
        plsc.addupdate_scatter(acc_v, [addr + (2 * B * L)], qv * pz)
        plsc.addupdate_scatter(acc_v, [addr + (3 * B * L)], px)
        plsc.addupdate_scatter(acc_v, [addr + (4 * B * L)], py)
        plsc.addupdate_scatter(acc_v, [addr + (5 * B * L)], pz)
        return qsum + qv

    bgrp = lax.iota(jnp.int32, L) * L
    for c in range(NCOMP):
        for g in range(B // L):
            a0 = c * B * L + g * L * L
            s = zero
            for l in range(L):
                s = s + plsc.load_gather(acc_v, [bgrp + (a0 + l)])
            res_v[pl.ds(c * B + g * L, L)] = s
    qs = jnp.broadcast_to(jnp.sum(qsum), (L,))
    res_v[pl.ds(NCOMP * B, L)] = qs
    res_v[pl.ds(NCOMP * B + L, L)] = qs
    res_v[pl.ds(NCOMP * B + 2 * L, L)] = qs
    res_v[pl.ds(NCOMP * B + 3 * L, L)] = qs
    res_v[pl.ds(7 * B, L)] = zero
    res_v[pl.ds(7 * B + L, L)] = zero
    res_v[pl.ds(7 * B + 2 * L, L)] = zero
    res_v[pl.ds(7 * B + 3 * L, L)] = zero

    pltpu.sync_copy(res_v, out_hbm.at[wid])


def _partials(pos, q, batch):
    mesh = plsc.VectorSubcoreMesh(
        core_axis_name="c", subcore_axis_name="s", num_cores=NC, num_subcores=NS
    )
    return pl.kernel(
        _tec_body,
        out_type=jax.ShapeDtypeStruct((NW, ROW), jnp.float32),
        mesh=mesh,
        scratch_types=[
            pltpu.VMEM((CHUNK,), jnp.float32),
            pltpu.VMEM((CHUNK,), jnp.float32),
            pltpu.VMEM((CHUNK,), jnp.float32),
            pltpu.VMEM((CHUNK,), jnp.float32),
            pltpu.VMEM((CHUNK,), jnp.int32),
            pltpu.VMEM((ACC_WORDS,), jnp.float32),
            pltpu.VMEM((ROW,), jnp.float32),
            pltpu.SemaphoreType.DMA,
            pltpu.SemaphoreType.DMA,
            pltpu.SemaphoreType.DMA,
            pltpu.SemaphoreType.DMA,
            pltpu.SemaphoreType.DMA,
        ],
        compiler_params=pltpu.CompilerParams(needs_layout_passes=False),
    )(pos, q, batch)


def kernel(positions, q, batch, cell):
    del cell
    pos_t = jnp.concatenate([positions[:, 0], positions[:, 1], positions[:, 2]])
    part = _partials(pos_t, q, batch.astype(jnp.int32))
    t = part.sum(axis=0).reshape(8, B)
    pol = (t[0:3] - (t[6] * (1.0 / N)) * t[3:6]).T * NORM
    return pol


# speedup vs baseline: 1.2242x; 1.2049x over previous
import jax
import jax.numpy as jnp
from jax import lax
from jax.experimental import pallas as pl
from jax.experimental.pallas import tpu as pltpu
from jax.experimental.pallas import tpu_sc as plsc

N = 524288
B = 64
NORM = 0.10538154

NC = 2
NS = 16
L = 16
NW = NC * NS
CHUNK = N // NW
NCOMP = 6
ACC_WORDS = NCOMP * B * L
ROW = 8 * B


def _tec_body(pos_hbm, q_hbm, batch_hbm, out_hbm,
              x_v, y_v, z_v, q_v, b_v, acc_v, res_v,
              sem0, sem1, sem2, sem3, sem4, sem5, sem6, sem7, sem8, sem9):
    cid = lax.axis_index("c")
    sid = lax.axis_index("s")
    wid = cid * NS + sid
    base = wid * CHUNK

    H = CHUNK // 2
    c0 = pltpu.async_copy(pos_hbm.at[pl.ds(base, H)], x_v.at[pl.ds(0, H)], sem0)
    c1 = pltpu.async_copy(pos_hbm.at[pl.ds(N + base, H)], y_v.at[pl.ds(0, H)], sem1)
    c2 = pltpu.async_copy(pos_hbm.at[pl.ds(2 * N + base, H)], z_v.at[pl.ds(0, H)], sem2)
    c3 = pltpu.async_copy(q_hbm.at[pl.ds(base, H)], q_v.at[pl.ds(0, H)], sem3)
    c4 = pltpu.async_copy(batch_hbm.at[pl.ds(base, H)], b_v.at[pl.ds(0, H)], sem4)
    d0 = pltpu.async_copy(pos_hbm.at[pl.ds(base + H, H)], x_v.at[pl.ds(H, H)], sem5)
    d1 = pltpu.async_copy(pos_hbm.at[pl.ds(N + base + H, H)], y_v.at[pl.ds(H, H)], sem6)
    d2 = pltpu.async_copy(pos_hbm.at[pl.ds(2 * N + base + H, H)], z_v.at[pl.ds(H, H)], sem7)
    d3 = pltpu.async_copy(q_hbm.at[pl.ds(base + H, H)], q_v.at[pl.ds(H, H)], sem8)
    d4 = pltpu.async_copy(batch_hbm.at[pl.ds(base + H, H)], b_v.at[pl.ds(H, H)], sem9)

    lane = lax.iota(jnp.int32, L)
    zero = jnp.zeros((L,), jnp.float32)

    def zbody(i, _):
        w = i * (4 * L)
        acc_v[pl.ds(w, L)] = zero
        acc_v[pl.ds(w + L, L)] = zero
        acc_v[pl.ds(w + 2 * L, L)] = zero
        acc_v[pl.ds(w + 3 * L, L)] = zero
        return 0
    lax.fori_loop(0, ACC_WORDS // (4 * L), zbody, 0)

    c0.wait()
    c1.wait()
    c2.wait()
    c3.wait()
    c4.wait()

    def step(el, qsum):
        px = x_v[pl.ds(el, L)]
        py = y_v[pl.ds(el, L)]
        pz = z_v[pl.ds(el, L)]
        qv = q_v[pl.ds(el, L)]
        bv = b_v[pl.ds(el, L)]
        addr = bv * L + lane
        plsc.addupdate_scatter(acc_v, [addr], qv * px)
        plsc.addupdate_scatter(acc_v, [addr + (B * L)], qv * py)
        plsc.addupdate_scatter(acc_v, [addr + (2 * B * L)], qv * pz)
        plsc.addupdate_scatter(acc_v, [addr + (3 * B * L)], px)
        plsc.addupdate_scatter(acc_v, [addr + (4 * B * L)], py)
        plsc.addupdate_scatter(acc_v, [addr + (5 * B * L)], pz)
        return qsum + qv

    qsum0 = plsc.parallel_loop(0, H, step=L, unroll=4, carry=zero)(step)

    d0.wait()
    d1.wait()
    d2.wait()
    d3.wait()
    d4.wait()

    qsum = plsc.parallel_loop(H, CHUNK, step=L, unroll=4, carry=qsum0)(step)

    bgrp = lax.iota(jnp.int32, L) * L
    for c in range(NCOMP):
        for g in range(B // L):
            a0 = c * B * L + g * L * L
            s = zero
            for l in range(L):
                s = s + plsc.load_gather(acc_v, [bgrp + (a0 + l)])
            res_v[pl.ds(c * B + g * L, L)] = s
    qs = jnp.broadcast_to(jnp.sum(qsum), (L,))
    res_v[pl.ds(NCOMP * B, L)] = qs
    res_v[pl.ds(NCOMP * B + L, L)] = qs
    res_v[pl.ds(NCOMP * B + 2 * L, L)] = qs
    res_v[pl.ds(NCOMP * B + 3 * L, L)] = qs
    res_v[pl.ds(7 * B, L)] = zero
    res_v[pl.ds(7 * B + L, L)] = zero
    res_v[pl.ds(7 * B + 2 * L, L)] = zero
    res_v[pl.ds(7 * B + 3 * L, L)] = zero

    pltpu.sync_copy(res_v, out_hbm.at[wid])


def _partials(pos, q, batch):
    mesh = plsc.VectorSubcoreMesh(
        core_axis_name="c", subcore_axis_name="s", num_cores=NC, num_subcores=NS
    )
    return pl.kernel(
        _tec_body,
        out_type=jax.ShapeDtypeStruct((NW, ROW), jnp.float32),
        mesh=mesh,
        scratch_types=[
            pltpu.VMEM((CHUNK,), jnp.float32),
            pltpu.VMEM((CHUNK,), jnp.float32),
            pltpu.VMEM((CHUNK,), jnp.float32),
            pltpu.VMEM((CHUNK,), jnp.float32),
            pltpu.VMEM((CHUNK,), jnp.int32),
            pltpu.VMEM((ACC_WORDS,), jnp.float32),
            pltpu.VMEM((ROW,), jnp.float32),
            pltpu.SemaphoreType.DMA,
            pltpu.SemaphoreType.DMA,
            pltpu.SemaphoreType.DMA,
            pltpu.SemaphoreType.DMA,
            pltpu.SemaphoreType.DMA,
            pltpu.SemaphoreType.DMA,
            pltpu.SemaphoreType.DMA,
            pltpu.SemaphoreType.DMA,
            pltpu.SemaphoreType.DMA,
            pltpu.SemaphoreType.DMA,
        ],
        compiler_params=pltpu.CompilerParams(needs_layout_passes=False),
    )(pos, q, batch)


def kernel(positions, q, batch, cell):
    del cell
    part = _partials(positions.T.reshape(3 * N), q, batch.astype(jnp.int32))
    t = part.sum(axis=0).reshape(8, B)
    pol = (t[0:3] - (t[6] * (1.0 / N)) * t[3:6]).T * NORM
    return pol
